# baseline (speedup 1.0000x reference)
import functools

import jax
import jax.numpy as jnp
from jax import lax
from jax.experimental import pallas as pl
from jax.experimental.pallas import tpu as pltpu
from jax.experimental.pallas import tpu_sc as plsc

N = 10000
N_PAD = 10240
E = 320000
EB = 128
R = E // EB
R_PAD = 2560
E_PAD = R_PAD * EB
RPT = R_PAD // 16
NPT = N_PAD // 16
SB = 16

_MESH = plsc.VectorSubcoreMesh(core_axis_name="c", subcore_axis_name="s")
_SC_PARAMS = pltpu.CompilerParams(needs_layout_passes=False,
                                  use_tc_tiling_on_sc=False)


def _f32(shape):
    return jax.ShapeDtypeStruct(shape, jnp.float32)


def _norm_body(row_h, col_h, w_h, norm_h, deg_s, dinv_s, colv, wv, rowv,
               normv, dinv_v, zb):
    cid = lax.axis_index("c")
    sid = lax.axis_index("s")
    del cid

    zv = jnp.zeros((16,), jnp.float32)

    def zloop(i, _):
        zb[pl.ds(i * 16, 16)] = zv
        return 0

    lax.fori_loop(0, NPT // 16, zloop, 0)
    pltpu.sync_copy(zb, deg_s.at[pl.ds(sid * NPT, NPT)])
    plsc.subcore_barrier()

    base = sid * RPT
    pltpu.sync_copy(col_h.at[pl.ds(base, RPT)], colv)
    pltpu.sync_copy(w_h.at[pl.ds(base, RPT)], wv)

    def dacc(j, _):
        pltpu.sync_copy(wv.at[j], deg_s.at[colv.at[j]], add=True)
        return 0

    lax.fori_loop(0, RPT, dacc, 0)
    plsc.subcore_barrier()

    nb = sid * NPT
    pltpu.sync_copy(deg_s.at[pl.ds(nb, NPT)], dinv_v.at[pl.ds(0, NPT)])

    def newton(i, _):
        d = dinv_v[pl.ds(i * 16, 16)]
        m = d > 0.0
        ds = jnp.where(m, d, 1.0)
        s = jnp.full((16,), 3.0, jnp.float32)
        for _unused in range(10):
            s = 0.5 * (s + ds / s)
        dinv_v[pl.ds(i * 16, 16)] = jnp.where(m, 1.0 / s, 0.0)
        return 0

    lax.fori_loop(0, NPT // 16, newton, 0)
    pltpu.sync_copy(dinv_v.at[pl.ds(0, NPT)], dinv_s.at[pl.ds(nb, NPT)])
    plsc.subcore_barrier()

    wid = sid * 2 + lax.axis_index("c")
    rows_w = R_PAD // 32
    eb = wid * rows_w
    pltpu.sync_copy(dinv_s, dinv_v)
    pltpu.sync_copy(row_h.at[pl.ds(eb, rows_w)], rowv)
    pltpu.sync_copy(col_h.at[pl.ds(eb, rows_w)], colv.at[pl.ds(0, rows_w)])
    pltpu.sync_copy(w_h.at[pl.ds(eb, rows_w)], wv.at[pl.ds(0, rows_w)])

    def crow(j, _):
        def c16(v, _2):
            r_idx = rowv[j, pl.ds(v * 16, 16)]
            c_idx = colv[j, pl.ds(v * 16, 16)]
            w16 = wv[j, pl.ds(v * 16, 16)]
            rd = plsc.load_gather(dinv_v, [r_idx])
            cd = plsc.load_gather(dinv_v, [c_idx])
            normv[j, pl.ds(v * 16, 16)] = rd * w16 * cd
            return 0

        lax.fori_loop(0, EB // 16, c16, 0)
        return 0

    lax.fori_loop(0, rows_w, crow, 0)
    pltpu.sync_copy(normv, norm_h.at[pl.ds(eb, rows_w)])


_norm_call = pl.kernel(
    _norm_body,
    out_type=_f32((R_PAD, EB)),
    mesh=_MESH,
    scratch_types=[
        pltpu.VMEM_SHARED((N_PAD,), jnp.float32),
        pltpu.VMEM_SHARED((N_PAD,), jnp.float32),
        pltpu.VMEM((RPT, EB), jnp.int32),
        pltpu.VMEM((RPT, EB), jnp.float32),
        pltpu.VMEM((R_PAD // 32, EB), jnp.int32),
        pltpu.VMEM((R_PAD // 32, EB), jnp.float32),
        pltpu.VMEM((N_PAD,), jnp.float32),
        pltpu.VMEM((NPT,), jnp.float32),
    ],
    compiler_params=_SC_PARAMS,
)


def _hop_body(nc, fc, horner, jg, h_in, row_h, col_h, norm_h, o1, o2, o3,
              acc, gb0, gb1, rowv, colv, normv, gs0, gs1, ss0, ss1):
    cid = lax.axis_index("c")
    sid = lax.axis_index("s")
    nch = nc // 2
    nvr = fc // 16
    outs = [o1, o2, o3]
    gbufs = (gb0, gb1)
    gsems = (gs0, gs1)
    ssems = (ss0, ss1)

    def start_gather(src, j, b):
        pltpu.async_copy(src.at[rowv.at[pl.ds(j * EB, EB)]],
                         gbufs[b], gsems[b])

    def wait_gather(src, b):
        pltpu.make_async_copy(src.at[pl.ds(0, EB)], gbufs[b],
                              gsems[b]).wait()

    def start_scatter(j, b):
        pltpu.async_copy(gbufs[b], acc.at[colv.at[j]], ssems[b], add=True)

    def wait_scatter(b):
        pltpu.make_async_copy(gbufs[b], acc.at[pl.ds(0, EB)],
                              ssems[b]).wait()

    def scale(j, b):
        gbuf = gbufs[b]

        def scale16(e16, _3):
            nv = normv[j, pl.ds(e16 * 16, 16)]
            for e in range(16):
                edge = e16 * 16 + e
                s = nv[e]
                for v in range(nvr):
                    g = gbuf[edge, pl.ds(v * 16, 16)]
                    gbuf[edge, pl.ds(v * 16, 16)] = g * s
            return 0

        lax.fori_loop(0, EB // 16, scale16, 0)

    def do_chunk(ci, _):
        c = cid * nch + ci
        for k in range(3):
            if horner:
                src = h_in.at[3 * nc + c] if k == 0 else outs[k - 1].at[c]
            else:
                src = h_in.at[c] if k == 0 else outs[k - 1].at[c]
            dst = outs[k]

            if horner:
                pltpu.sync_copy(
                    h_in.at[(2 - k) * nc + c].at[pl.ds(sid * NPT, NPT)],
                    acc.at[pl.ds(sid * NPT, NPT)])
            else:
                zv = jnp.zeros((16,), jnp.float32)
                zbuf = gb0 if jg == 1 else gb0.at[pl.ds(0, EB)]

                def zrow(r, _2):
                    for v in range(nvr):
                        zbuf[r, pl.ds(v * 16, 16)] = zv
                    return 0

                lax.fori_loop(0, EB, zrow, 0)
                for i in range(NPT // EB):
                    pltpu.sync_copy(zbuf,
                                    acc.at[pl.ds(sid * NPT + i * EB, EB)])
            plsc.subcore_barrier()

            base = sid * RPT

            def sub_block(sb, _1):
                off = base + sb * SB
                pltpu.sync_copy(row_h.at[pl.ds(off * EB, SB * EB)], rowv)
                pltpu.sync_copy(col_h.at[pl.ds(off, SB)], colv)
                pltpu.sync_copy(norm_h.at[pl.ds(off, SB)], normv)
                if jg == 1:
                    start_gather(src, 0, 0)
                    start_gather(src, 1, 1)

                    def pair(p, _2):
                        for b in range(2):
                            j = p * 2 + b
                            wait_gather(src, b)
                            scale(j, b)
                            start_scatter(j, b)

                        @pl.when(p < SB // 2 - 1)
                        def _prefetch():
                            for b in range(2):
                                wait_scatter(b)
                                start_gather(src, p * 2 + 2 + b, b)

                        return 0

                    lax.fori_loop(0, SB // 2, pair, 0)
                    wait_scatter(0)
                    wait_scatter(1)
                else:
                    def group(q, _2):
                        qj = q * jg
                        pltpu.sync_copy(
                            src.at[rowv.at[pl.ds(qj * EB, jg * EB)]], gb0)

                        def scale16g(e16, _3):
                            for jj in range(jg):
                                nv = normv[qj + jj, pl.ds(e16 * 16, 16)]
                                for e in range(16):
                                    edge = jj * EB + e16 * 16 + e
                                    s = nv[e]
                                    for v in range(nvr):
                                        g = gb0[edge, pl.ds(v * 16, 16)]
                                        gb0[edge, pl.ds(v * 16, 16)] = g * s
                            return 0

                        lax.fori_loop(0, EB // 16, scale16g, 0)
                        for jj in range(jg):
                            pltpu.sync_copy(gb0.at[pl.ds(jj * EB, EB)],
                                            acc.at[colv.at[qj + jj]],
                                            add=True)
                        return 0

                    lax.fori_loop(0, SB // jg, group, 0)
                return 0

            lax.fori_loop(0, RPT // SB, sub_block, 0)
            plsc.subcore_barrier()
            pltpu.sync_copy(acc.at[pl.ds(sid * NPT, NPT)],
                            dst.at[c].at[pl.ds(sid * NPT, NPT)])
            plsc.subcore_barrier()
        return 0

    lax.fori_loop(0, nch, do_chunk, 0)


def _make_hop_call(nc, fc, horner=False, jg=1):
    body = functools.partial(_hop_body, nc, fc, horner, jg)
    gb0_shape = (EB, fc) if jg == 1 else (jg * EB, fc)
    gb1_shape = (EB, fc) if jg == 1 else (16,)
    return pl.kernel(
        body,
        out_type=[_f32((nc, N_PAD, fc))] * 3,
        mesh=_MESH,
        scratch_types=[
            pltpu.VMEM_SHARED((N_PAD, fc), jnp.float32),
            pltpu.VMEM(gb0_shape, jnp.float32),
            pltpu.VMEM(gb1_shape, jnp.float32),
            pltpu.VMEM((SB * EB,), jnp.int32),
            pltpu.VMEM((SB, EB), jnp.int32),
            pltpu.VMEM((SB, EB), jnp.float32),
            pltpu.SemaphoreType.DMA,
            pltpu.SemaphoreType.DMA,
            pltpu.SemaphoreType.DMA,
            pltpu.SemaphoreType.DMA,
        ],
        compiler_params=_SC_PARAMS,
    )


_hop_l1 = _make_hop_call(2, 64)
_hop_l2 = _make_hop_call(4, 128, jg=2)
_hop_l3 = _make_hop_call(2, 32, horner=True)


def _mm_body(n_in, nc_out, elu, *refs):
    h_refs = refs[:4]
    w_ref, b_ref, out_ref = refs[4], refs[5], refs[6]
    nch_in = n_in // 4
    acc = jnp.broadcast_to(b_ref[...], (256, b_ref.shape[1])).astype(jnp.float32)
    for k in range(4):
        for c in range(nch_in):
            acc = acc + jnp.dot(h_refs[k][c], w_ref[k * nch_in + c],
                                preferred_element_type=jnp.float32)
    if elu:
        acc = jnp.where(acc > 0, acc, jnp.exp(acc) - 1.0)
    if nc_out > 1:
        fc = acc.shape[1] // nc_out
        for c in range(nc_out):
            out_ref[c] = acc[:, c * fc:(c + 1) * fc]
    else:
        out_ref[...] = acc


def _make_mm_call(nc_in, fc_in, h_out, nc_out, elu):
    n_in = 4 * nc_in
    body = functools.partial(_mm_body, n_in, nc_out, elu)
    if nc_out > 1:
        out_shape = _f32((nc_out, N_PAD, h_out // nc_out))
        out_spec = pl.BlockSpec((nc_out, 256, h_out // nc_out),
                                lambda m: (0, m, 0))
    else:
        out_shape = _f32((N_PAD, h_out))
        out_spec = pl.BlockSpec((256, h_out), lambda m: (m, 0))
    h_spec = pl.BlockSpec((nc_in, 256, fc_in), lambda m: (0, m, 0))
    return pl.pallas_call(
        body,
        grid=(N_PAD // 256,),
        in_specs=[h_spec] * 4 + [
            pl.BlockSpec((n_in, fc_in, h_out), lambda m: (0, 0, 0)),
            pl.BlockSpec((1, h_out), lambda m: (0, 0)),
        ],
        out_specs=out_spec,
        out_shape=out_shape,
    )


_mm_l1 = _make_mm_call(2, 64, 512, 4, True)
_mm_l2 = _make_mm_call(4, 128, 512, 4, True)


def _mm3_body(h_ref, w_ref, b_ref, out_ref):
    acc = jnp.broadcast_to(b_ref[...], (256, 256)).astype(jnp.float32)
    for c in range(4):
        acc = acc + jnp.dot(h_ref[c], w_ref[c],
                            preferred_element_type=jnp.float32)
    for q in range(8):
        k, cc = q // 2, q % 2
        out_ref[q] = acc[:, k * 64 + cc * 32: k * 64 + cc * 32 + 32]


_mm_l3 = pl.pallas_call(
    _mm3_body,
    grid=(N_PAD // 256,),
    in_specs=[
        pl.BlockSpec((4, 256, 128), lambda m: (0, m, 0)),
        pl.BlockSpec((4, 128, 256), lambda m: (0, 0, 0)),
        pl.BlockSpec((1, 256), lambda m: (0, 0)),
    ],
    out_specs=pl.BlockSpec((8, 256, 32), lambda m: (0, m, 0)),
    out_shape=_f32((8, N_PAD, 32)),
)


def kernel(x, edge_index, weight, W1, b1, W2, b2, W3, b3):
    f32 = jnp.float32
    i32 = jnp.int32
    pad_e = E_PAD - E
    row_f = jnp.concatenate([edge_index[0], jnp.zeros((pad_e,), i32)])
    row2d = row_f.reshape(R_PAD, EB)
    col2d = jnp.concatenate([edge_index[1], jnp.zeros((pad_e,), i32)]
                            ).reshape(R_PAD, EB)
    w2d = jnp.concatenate([weight, jnp.zeros((pad_e,), f32)]
                          ).reshape(R_PAD, EB)

    norm2d = _norm_call(row2d, col2d, w2d)

    xp = jnp.pad(x, ((0, N_PAD - N), (0, 0)))
    x3 = xp.reshape(N_PAD, 2, 64).transpose(1, 0, 2)

    h1, h2, h3 = _hop_l1(x3, row_f, col2d, norm2d)
    w1b = W1.reshape(4, 2, 64, 512).reshape(8, 64, 512)
    y1 = _mm_l1(x3, h1, h2, h3, w1b, b1.reshape(1, 512))

    h1, h2, h3 = _hop_l2(y1, row_f, col2d, norm2d)
    w2b = W2.reshape(4, 4, 128, 512).reshape(16, 128, 512)
    y2 = _mm_l2(y1, h1, h2, h3, w2b, b2.reshape(1, 512))

    w3p = jnp.pad(W3, ((0, 0), (0, 0), (0, 64 - 40)))
    w3cat = w3p.transpose(1, 0, 2).reshape(512, 256).reshape(4, 128, 256)
    b3cat = jnp.concatenate([jnp.pad(b3, (0, 64 - 40)),
                             jnp.zeros((192,), f32)]).reshape(1, 256)
    p = _mm_l3(y2, w3cat, b3cat)
    _, _, out = _hop_l3(p, row_f, col2d, norm2d)
    y3 = jnp.concatenate([out[0], out[1]], axis=1)
    return y3[:N, :40]


# speedup vs baseline: 1.1084x; 1.1084x over previous
import functools

import jax
import jax.numpy as jnp
from jax import lax
from jax.experimental import pallas as pl
from jax.experimental.pallas import tpu as pltpu
from jax.experimental.pallas import tpu_sc as plsc

N = 10000
N_PAD = 10240
E = 320000
EB = 128
R = E // EB
R_PAD = 2560
E_PAD = R_PAD * EB
RPT = R_PAD // 16
NPT = N_PAD // 16
SB = 16

_MESH = plsc.VectorSubcoreMesh(core_axis_name="c", subcore_axis_name="s")
_SC_PARAMS = pltpu.CompilerParams(needs_layout_passes=False,
                                  use_tc_tiling_on_sc=False)


def _f32(shape):
    return jax.ShapeDtypeStruct(shape, jnp.float32)


def _norm_body(row_h, col_h, w_h, norm_h, deg_s, dinv_s, colv, wv, rowv,
               normv, dinv_v, zb):
    cid = lax.axis_index("c")
    sid = lax.axis_index("s")
    del cid

    zv = jnp.zeros((16,), jnp.float32)

    def zloop(i, _):
        zb[pl.ds(i * 16, 16)] = zv
        return 0

    lax.fori_loop(0, NPT // 16, zloop, 0)
    pltpu.sync_copy(zb, deg_s.at[pl.ds(sid * NPT, NPT)])
    plsc.subcore_barrier()

    base = sid * RPT
    pltpu.sync_copy(col_h.at[pl.ds(base, RPT)], colv)
    pltpu.sync_copy(w_h.at[pl.ds(base, RPT)], wv)

    def dacc(j, _):
        pltpu.sync_copy(wv.at[j], deg_s.at[colv.at[j]], add=True)
        return 0

    lax.fori_loop(0, RPT, dacc, 0)
    plsc.subcore_barrier()

    nb = sid * NPT
    pltpu.sync_copy(deg_s.at[pl.ds(nb, NPT)], dinv_v.at[pl.ds(0, NPT)])

    def newton(i, _):
        d = dinv_v[pl.ds(i * 16, 16)]
        m = d > 0.0
        ds = jnp.where(m, d, 1.0)
        s = jnp.full((16,), 3.0, jnp.float32)
        for _unused in range(10):
            s = 0.5 * (s + ds / s)
        dinv_v[pl.ds(i * 16, 16)] = jnp.where(m, 1.0 / s, 0.0)
        return 0

    lax.fori_loop(0, NPT // 16, newton, 0)
    pltpu.sync_copy(dinv_v.at[pl.ds(0, NPT)], dinv_s.at[pl.ds(nb, NPT)])
    plsc.subcore_barrier()

    wid = sid * 2 + lax.axis_index("c")
    rows_w = R_PAD // 32
    eb = wid * rows_w
    pltpu.sync_copy(dinv_s, dinv_v)
    pltpu.sync_copy(row_h.at[pl.ds(eb, rows_w)], rowv)
    pltpu.sync_copy(col_h.at[pl.ds(eb, rows_w)], colv.at[pl.ds(0, rows_w)])
    pltpu.sync_copy(w_h.at[pl.ds(eb, rows_w)], wv.at[pl.ds(0, rows_w)])

    def crow(j, _):
        def c16(v, _2):
            r_idx = rowv[j, pl.ds(v * 16, 16)]
            c_idx = colv[j, pl.ds(v * 16, 16)]
            w16 = wv[j, pl.ds(v * 16, 16)]
            rd = plsc.load_gather(dinv_v, [r_idx])
            cd = plsc.load_gather(dinv_v, [c_idx])
            normv[j, pl.ds(v * 16, 16)] = rd * w16 * cd
            return 0

        lax.fori_loop(0, EB // 16, c16, 0)
        return 0

    lax.fori_loop(0, rows_w, crow, 0)
    pltpu.sync_copy(normv, norm_h.at[pl.ds(eb, rows_w)])


_norm_call = pl.kernel(
    _norm_body,
    out_type=_f32((R_PAD, EB)),
    mesh=_MESH,
    scratch_types=[
        pltpu.VMEM_SHARED((N_PAD,), jnp.float32),
        pltpu.VMEM_SHARED((N_PAD,), jnp.float32),
        pltpu.VMEM((RPT, EB), jnp.int32),
        pltpu.VMEM((RPT, EB), jnp.float32),
        pltpu.VMEM((R_PAD // 32, EB), jnp.int32),
        pltpu.VMEM((R_PAD // 32, EB), jnp.float32),
        pltpu.VMEM((N_PAD,), jnp.float32),
        pltpu.VMEM((NPT,), jnp.float32),
    ],
    compiler_params=_SC_PARAMS,
)


def _hop_body(nc, fc, horner, jg, h_in, row_h, col_h, norm_h, o1, o2, o3,
              acc, gb0, gb1, rowv, colv, normv, gs0, gs1, ss0, ss1):
    cid = lax.axis_index("c")
    sid = lax.axis_index("s")
    nch = nc // 2
    nvr = fc // 16
    outs = [o1, o2, o3]
    gbufs = (gb0, gb1)
    gsems = (gs0, gs1)
    ssems = (ss0, ss1)

    def start_gather(src, j, b):
        pltpu.async_copy(src.at[rowv.at[j]], gbufs[b], gsems[b])

    def wait_gather(src, b):
        pltpu.make_async_copy(src.at[pl.ds(0, EB)], gbufs[b],
                              gsems[b]).wait()

    def start_scatter(j, b):
        pltpu.async_copy(gbufs[b], acc.at[colv.at[j]], ssems[b], add=True)

    def wait_scatter(b):
        pltpu.make_async_copy(gbufs[b], acc.at[pl.ds(0, EB)],
                              ssems[b]).wait()

    def scale(j, b):
        gbuf = gbufs[b]

        def scale16(e16, _3):
            nv = normv[j, pl.ds(e16 * 16, 16)]
            for e in range(16):
                edge = e16 * 16 + e
                s = nv[e]
                for v in range(nvr):
                    g = gbuf[edge, pl.ds(v * 16, 16)]
                    gbuf[edge, pl.ds(v * 16, 16)] = g * s
            return 0

        lax.fori_loop(0, EB // 16, scale16, 0)

    def do_chunk(ci, _):
        c = cid * nch + ci
        for k in range(3):
            if horner:
                src = h_in.at[3 * nc + c] if k == 0 else outs[k - 1].at[c]
            else:
                src = h_in.at[c] if k == 0 else outs[k - 1].at[c]
            dst = outs[k]

            if horner:
                pltpu.sync_copy(
                    h_in.at[(2 - k) * nc + c].at[pl.ds(sid * NPT, NPT)],
                    acc.at[pl.ds(sid * NPT, NPT)])
            else:
                zv = jnp.zeros((16,), jnp.float32)
                zbuf = gb0 if jg == 1 else gb0.at[0]

                def zrow(r, _2):
                    for v in range(nvr):
                        zbuf[r, pl.ds(v * 16, 16)] = zv
                    return 0

                lax.fori_loop(0, EB, zrow, 0)
                for i in range(NPT // EB):
                    pltpu.sync_copy(zbuf,
                                    acc.at[pl.ds(sid * NPT + i * EB, EB)])
            plsc.subcore_barrier()

            base = sid * RPT

            def sub_block(sb, _1):
                off = base + sb * SB
                pltpu.sync_copy(row_h.at[pl.ds(off, SB)], rowv)
                pltpu.sync_copy(col_h.at[pl.ds(off, SB)], colv)
                pltpu.sync_copy(norm_h.at[pl.ds(off, SB)], normv)
                if jg == 1:
                    start_gather(src, 0, 0)
                    start_gather(src, 1, 1)

                    def pair(p, _2):
                        for b in range(2):
                            j = p * 2 + b
                            wait_gather(src, b)
                            scale(j, b)
                            start_scatter(j, b)

                        @pl.when(p < SB // 2 - 1)
                        def _prefetch():
                            for b in range(2):
                                wait_scatter(b)
                                start_gather(src, p * 2 + 2 + b, b)

                        return 0

                    lax.fori_loop(0, SB // 2, pair, 0)
                    wait_scatter(0)
                    wait_scatter(1)
                else:
                    def group(q, _2):
                        qj = q * jg
                        pltpu.sync_copy(src.at[rowv.at[pl.ds(qj, jg)]], gb0)

                        def scale16g(e16, _3):
                            for jj in range(jg):
                                nv = normv[qj + jj, pl.ds(e16 * 16, 16)]
                                for e in range(16):
                                    edge = e16 * 16 + e
                                    s = nv[e]
                                    for v in range(nvr):
                                        g = gb0[jj, edge, pl.ds(v * 16, 16)]
                                        gb0[jj, edge,
                                            pl.ds(v * 16, 16)] = g * s
                            return 0

                        lax.fori_loop(0, EB // 16, scale16g, 0)
                        pltpu.sync_copy(gb0, acc.at[colv.at[pl.ds(qj, jg)]],
                                        add=True)
                        return 0

                    lax.fori_loop(0, SB // jg, group, 0)
                return 0

            lax.fori_loop(0, RPT // SB, sub_block, 0)
            plsc.subcore_barrier()
            pltpu.sync_copy(acc.at[pl.ds(sid * NPT, NPT)],
                            dst.at[c].at[pl.ds(sid * NPT, NPT)])
            plsc.subcore_barrier()
        return 0

    lax.fori_loop(0, nch, do_chunk, 0)


def _make_hop_call(nc, fc, horner=False, jg=1):
    body = functools.partial(_hop_body, nc, fc, horner, jg)
    gb0_shape = (EB, fc) if jg == 1 else (jg, EB, fc)
    gb1_shape = (EB, fc) if jg == 1 else (16,)
    return pl.kernel(
        body,
        out_type=[_f32((nc, N_PAD, fc))] * 3,
        mesh=_MESH,
        scratch_types=[
            pltpu.VMEM_SHARED((N_PAD, fc), jnp.float32),
            pltpu.VMEM(gb0_shape, jnp.float32),
            pltpu.VMEM(gb1_shape, jnp.float32),
            pltpu.VMEM((SB, EB), jnp.int32),
            pltpu.VMEM((SB, EB), jnp.int32),
            pltpu.VMEM((SB, EB), jnp.float32),
            pltpu.SemaphoreType.DMA,
            pltpu.SemaphoreType.DMA,
            pltpu.SemaphoreType.DMA,
            pltpu.SemaphoreType.DMA,
        ],
        compiler_params=_SC_PARAMS,
    )


_hop_l1 = _make_hop_call(2, 64)
_hop_l2 = _make_hop_call(4, 128)
_hop_l3 = _make_hop_call(2, 32, horner=True)


def _mm_body(n_in, nc_out, elu, *refs):
    h_refs = refs[:4]
    w_ref, b_ref, out_ref = refs[4], refs[5], refs[6]
    nch_in = n_in // 4
    acc = jnp.broadcast_to(b_ref[...], (256, b_ref.shape[1])).astype(jnp.float32)
    for k in range(4):
        for c in range(nch_in):
            acc = acc + jnp.dot(h_refs[k][c], w_ref[k * nch_in + c],
                                preferred_element_type=jnp.float32)
    if elu:
        acc = jnp.where(acc > 0, acc, jnp.exp(acc) - 1.0)
    if nc_out > 1:
        fc = acc.shape[1] // nc_out
        for c in range(nc_out):
            out_ref[c] = acc[:, c * fc:(c + 1) * fc]
    else:
        out_ref[...] = acc


def _make_mm_call(nc_in, fc_in, h_out, nc_out, elu):
    n_in = 4 * nc_in
    body = functools.partial(_mm_body, n_in, nc_out, elu)
    if nc_out > 1:
        out_shape = _f32((nc_out, N_PAD, h_out // nc_out))
        out_spec = pl.BlockSpec((nc_out, 256, h_out // nc_out),
                                lambda m: (0, m, 0))
    else:
        out_shape = _f32((N_PAD, h_out))
        out_spec = pl.BlockSpec((256, h_out), lambda m: (m, 0))
    h_spec = pl.BlockSpec((nc_in, 256, fc_in), lambda m: (0, m, 0))
    return pl.pallas_call(
        body,
        grid=(N_PAD // 256,),
        in_specs=[h_spec] * 4 + [
            pl.BlockSpec((n_in, fc_in, h_out), lambda m: (0, 0, 0)),
            pl.BlockSpec((1, h_out), lambda m: (0, 0)),
        ],
        out_specs=out_spec,
        out_shape=out_shape,
    )


_mm_l1 = _make_mm_call(2, 64, 512, 4, True)
_mm_l2 = _make_mm_call(4, 128, 512, 4, True)


def _mm3_body(h_ref, w_ref, b_ref, out_ref):
    acc = jnp.broadcast_to(b_ref[...], (256, 256)).astype(jnp.float32)
    for c in range(4):
        acc = acc + jnp.dot(h_ref[c], w_ref[c],
                            preferred_element_type=jnp.float32)
    for q in range(8):
        k, cc = q // 2, q % 2
        out_ref[q] = acc[:, k * 64 + cc * 32: k * 64 + cc * 32 + 32]


_mm_l3 = pl.pallas_call(
    _mm3_body,
    grid=(N_PAD // 256,),
    in_specs=[
        pl.BlockSpec((4, 256, 128), lambda m: (0, m, 0)),
        pl.BlockSpec((4, 128, 256), lambda m: (0, 0, 0)),
        pl.BlockSpec((1, 256), lambda m: (0, 0)),
    ],
    out_specs=pl.BlockSpec((8, 256, 32), lambda m: (0, m, 0)),
    out_shape=_f32((8, N_PAD, 32)),
)


def kernel(x, edge_index, weight, W1, b1, W2, b2, W3, b3):
    f32 = jnp.float32
    i32 = jnp.int32
    pad_e = E_PAD - E
    row2d = jnp.concatenate([edge_index[0], jnp.zeros((pad_e,), i32)]
                            ).reshape(R_PAD, EB)
    col2d = jnp.concatenate([edge_index[1], jnp.zeros((pad_e,), i32)]
                            ).reshape(R_PAD, EB)
    w2d = jnp.concatenate([weight, jnp.zeros((pad_e,), f32)]
                          ).reshape(R_PAD, EB)

    norm2d = _norm_call(row2d, col2d, w2d)

    xp = jnp.pad(x, ((0, N_PAD - N), (0, 0)))
    x3 = xp.reshape(N_PAD, 2, 64).transpose(1, 0, 2)

    h1, h2, h3 = _hop_l1(x3, row2d, col2d, norm2d)
    w1b = W1.reshape(4, 2, 64, 512).reshape(8, 64, 512)
    y1 = _mm_l1(x3, h1, h2, h3, w1b, b1.reshape(1, 512))

    h1, h2, h3 = _hop_l2(y1, row2d, col2d, norm2d)
    w2b = W2.reshape(4, 4, 128, 512).reshape(16, 128, 512)
    y2 = _mm_l2(y1, h1, h2, h3, w2b, b2.reshape(1, 512))

    w3p = jnp.pad(W3, ((0, 0), (0, 0), (0, 64 - 40)))
    w3cat = w3p.transpose(1, 0, 2).reshape(512, 256).reshape(4, 128, 256)
    b3cat = jnp.concatenate([jnp.pad(b3, (0, 64 - 40)),
                             jnp.zeros((192,), f32)]).reshape(1, 256)
    p = _mm_l3(y2, w3cat, b3cat)
    _, _, out = _hop_l3(p, row2d, col2d, norm2d)
    y3 = jnp.concatenate([out[0], out[1]], axis=1)
    return y3[:N, :40]


# speedup vs baseline: 1.1141x; 1.0051x over previous
import functools

import jax
import jax.numpy as jnp
from jax import lax
from jax.experimental import pallas as pl
from jax.experimental.pallas import tpu as pltpu
from jax.experimental.pallas import tpu_sc as plsc

N = 10000
N_PAD = 10240
E = 320000
EB = 128
R = E // EB
R_PAD = 2560
E_PAD = R_PAD * EB
RPT = R_PAD // 16
NPT = N_PAD // 16
SB = 16

_MESH = plsc.VectorSubcoreMesh(core_axis_name="c", subcore_axis_name="s")
_SC_PARAMS = pltpu.CompilerParams(needs_layout_passes=False,
                                  use_tc_tiling_on_sc=False)


def _f32(shape):
    return jax.ShapeDtypeStruct(shape, jnp.float32)


def _norm_body(row_h, col_h, w_h, norm_h, deg_s, dinv_s, colv, wv, rowv,
               normv, dinv_v, zb):
    cid = lax.axis_index("c")
    sid = lax.axis_index("s")
    del cid

    zv = jnp.zeros((16,), jnp.float32)

    def zloop(i, _):
        zb[pl.ds(i * 16, 16)] = zv
        return 0

    lax.fori_loop(0, NPT // 16, zloop, 0)
    pltpu.sync_copy(zb, deg_s.at[pl.ds(sid * NPT, NPT)])
    plsc.subcore_barrier()

    base = sid * RPT
    pltpu.sync_copy(col_h.at[pl.ds(base, RPT)], colv)
    pltpu.sync_copy(w_h.at[pl.ds(base, RPT)], wv)

    def dacc(j, _):
        pltpu.sync_copy(wv.at[j], deg_s.at[colv.at[j]], add=True)
        return 0

    lax.fori_loop(0, RPT, dacc, 0)
    plsc.subcore_barrier()

    nb = sid * NPT
    pltpu.sync_copy(deg_s.at[pl.ds(nb, NPT)], dinv_v.at[pl.ds(0, NPT)])

    def newton(i, _):
        d = dinv_v[pl.ds(i * 16, 16)]
        m = d > 0.0
        ds = jnp.where(m, d, 1.0)
        s = jnp.full((16,), 3.0, jnp.float32)
        for _unused in range(10):
            s = 0.5 * (s + ds / s)
        dinv_v[pl.ds(i * 16, 16)] = jnp.where(m, 1.0 / s, 0.0)
        return 0

    lax.fori_loop(0, NPT // 16, newton, 0)
    pltpu.sync_copy(dinv_v.at[pl.ds(0, NPT)], dinv_s.at[pl.ds(nb, NPT)])
    plsc.subcore_barrier()

    wid = sid * 2 + lax.axis_index("c")
    rows_w = R_PAD // 32
    eb = wid * rows_w
    pltpu.sync_copy(dinv_s, dinv_v)
    pltpu.sync_copy(row_h.at[pl.ds(eb, rows_w)], rowv)
    pltpu.sync_copy(col_h.at[pl.ds(eb, rows_w)], colv.at[pl.ds(0, rows_w)])
    pltpu.sync_copy(w_h.at[pl.ds(eb, rows_w)], wv.at[pl.ds(0, rows_w)])

    def crow(j, _):
        def c16(v, _2):
            r_idx = rowv[j, pl.ds(v * 16, 16)]
            c_idx = colv[j, pl.ds(v * 16, 16)]
            w16 = wv[j, pl.ds(v * 16, 16)]
            rd = plsc.load_gather(dinv_v, [r_idx])
            cd = plsc.load_gather(dinv_v, [c_idx])
            normv[j, pl.ds(v * 16, 16)] = rd * w16 * cd
            return 0

        lax.fori_loop(0, EB // 16, c16, 0)
        return 0

    lax.fori_loop(0, rows_w, crow, 0)
    pltpu.sync_copy(normv, norm_h.at[pl.ds(eb, rows_w)])


_norm_call = pl.kernel(
    _norm_body,
    out_type=_f32((R_PAD, EB)),
    mesh=_MESH,
    scratch_types=[
        pltpu.VMEM_SHARED((N_PAD,), jnp.float32),
        pltpu.VMEM_SHARED((N_PAD,), jnp.float32),
        pltpu.VMEM((RPT, EB), jnp.int32),
        pltpu.VMEM((RPT, EB), jnp.float32),
        pltpu.VMEM((R_PAD // 32, EB), jnp.int32),
        pltpu.VMEM((R_PAD // 32, EB), jnp.float32),
        pltpu.VMEM((N_PAD,), jnp.float32),
        pltpu.VMEM((NPT,), jnp.float32),
    ],
    compiler_params=_SC_PARAMS,
)


def _hop_body(nc, fc, horner, nb, h_in, row_h, col_h, norm_h, o1, o2, o3,
              acc, rowv, colv, normv, *bufsems):
    cid = lax.axis_index("c")
    sid = lax.axis_index("s")
    nch = nc // 2
    nvr = fc // 16
    outs = [o1, o2, o3]
    gbufs = bufsems[:nb]
    gsems = bufsems[nb:2 * nb]
    ssems = bufsems[2 * nb:3 * nb]

    def start_gather(src, j, b):
        pltpu.async_copy(src.at[rowv.at[j]], gbufs[b], gsems[b])

    def wait_gather(src, b):
        pltpu.make_async_copy(src.at[pl.ds(0, EB)], gbufs[b],
                              gsems[b]).wait()

    def start_scatter(j, b):
        pltpu.async_copy(gbufs[b], acc.at[colv.at[j]], ssems[b], add=True)

    def wait_scatter(b):
        pltpu.make_async_copy(gbufs[b], acc.at[pl.ds(0, EB)],
                              ssems[b]).wait()

    def scale(j, b):
        gbuf = gbufs[b]

        def scale16(e16, _3):
            nv = normv[j, pl.ds(e16 * 16, 16)]
            for e in range(16):
                edge = e16 * 16 + e
                s = nv[e]
                for v in range(nvr):
                    g = gbuf[edge, pl.ds(v * 16, 16)]
                    gbuf[edge, pl.ds(v * 16, 16)] = g * s
            return 0

        lax.fori_loop(0, EB // 16, scale16, 0)

    def do_chunk(ci, _):
        c = cid * nch + ci
        for k in range(3):
            if horner:
                src = h_in.at[3 * nc + c] if k == 0 else outs[k - 1].at[c]
            else:
                src = h_in.at[c] if k == 0 else outs[k - 1].at[c]
            dst = outs[k]

            if horner:
                pltpu.sync_copy(
                    h_in.at[(2 - k) * nc + c].at[pl.ds(sid * NPT, NPT)],
                    acc.at[pl.ds(sid * NPT, NPT)])
            else:
                zv = jnp.zeros((16,), jnp.float32)
                zbuf = gbufs[0]

                def zrow(r, _2):
                    for v in range(nvr):
                        zbuf[r, pl.ds(v * 16, 16)] = zv
                    return 0

                lax.fori_loop(0, EB, zrow, 0)
                for i in range(NPT // EB):
                    pltpu.sync_copy(zbuf,
                                    acc.at[pl.ds(sid * NPT + i * EB, EB)])
            plsc.subcore_barrier()

            base = sid * RPT

            def sub_block(sb, _1):
                off = base + sb * SB
                pltpu.sync_copy(row_h.at[pl.ds(off, SB)], rowv)
                pltpu.sync_copy(col_h.at[pl.ds(off, SB)], colv)
                pltpu.sync_copy(norm_h.at[pl.ds(off, SB)], normv)
                for b in range(nb):
                    start_gather(src, b, b)

                def group(p, _2):
                    for b in range(nb):
                        j = p * nb + b
                        wait_gather(src, b)
                        scale(j, b)
                        start_scatter(j, b)

                    @pl.when(p < SB // nb - 1)
                    def _prefetch():
                        for b in range(nb):
                            wait_scatter(b)
                            start_gather(src, p * nb + nb + b, b)

                    return 0

                lax.fori_loop(0, SB // nb, group, 0)
                for b in range(nb):
                    wait_scatter(b)
                return 0

            lax.fori_loop(0, RPT // SB, sub_block, 0)
            plsc.subcore_barrier()
            pltpu.sync_copy(acc.at[pl.ds(sid * NPT, NPT)],
                            dst.at[c].at[pl.ds(sid * NPT, NPT)])
            plsc.subcore_barrier()
        return 0

    lax.fori_loop(0, nch, do_chunk, 0)


def _make_hop_call(nc, fc, horner=False, nb=2):
    body = functools.partial(_hop_body, nc, fc, horner, nb)
    return pl.kernel(
        body,
        out_type=[_f32((nc, N_PAD, fc))] * 3,
        mesh=_MESH,
        scratch_types=[
            pltpu.VMEM_SHARED((N_PAD, fc), jnp.float32),
            pltpu.VMEM((SB, EB), jnp.int32),
            pltpu.VMEM((SB, EB), jnp.int32),
            pltpu.VMEM((SB, EB), jnp.float32),
        ] + [pltpu.VMEM((EB, fc), jnp.float32)] * nb
          + [pltpu.SemaphoreType.DMA] * (2 * nb),
        compiler_params=_SC_PARAMS,
    )


_hop_l1 = _make_hop_call(2, 64, nb=4)
_hop_l2 = _make_hop_call(4, 128, nb=2)
_hop_l3 = _make_hop_call(2, 32, horner=True, nb=4)


def _mm_body(n_in, nc_out, elu, *refs):
    h_refs = refs[:4]
    w_ref, b_ref, out_ref = refs[4], refs[5], refs[6]
    nch_in = n_in // 4
    acc = jnp.broadcast_to(b_ref[...], (256, b_ref.shape[1])).astype(jnp.float32)
    for k in range(4):
        for c in range(nch_in):
            acc = acc + jnp.dot(h_refs[k][c], w_ref[k * nch_in + c],
                                preferred_element_type=jnp.float32)
    if elu:
        acc = jnp.where(acc > 0, acc, jnp.exp(acc) - 1.0)
    if nc_out > 1:
        fc = acc.shape[1] // nc_out
        for c in range(nc_out):
            out_ref[c] = acc[:, c * fc:(c + 1) * fc]
    else:
        out_ref[...] = acc


def _make_mm_call(nc_in, fc_in, h_out, nc_out, elu):
    n_in = 4 * nc_in
    body = functools.partial(_mm_body, n_in, nc_out, elu)
    if nc_out > 1:
        out_shape = _f32((nc_out, N_PAD, h_out // nc_out))
        out_spec = pl.BlockSpec((nc_out, 256, h_out // nc_out),
                                lambda m: (0, m, 0))
    else:
        out_shape = _f32((N_PAD, h_out))
        out_spec = pl.BlockSpec((256, h_out), lambda m: (m, 0))
    h_spec = pl.BlockSpec((nc_in, 256, fc_in), lambda m: (0, m, 0))
    return pl.pallas_call(
        body,
        grid=(N_PAD // 256,),
        in_specs=[h_spec] * 4 + [
            pl.BlockSpec((n_in, fc_in, h_out), lambda m: (0, 0, 0)),
            pl.BlockSpec((1, h_out), lambda m: (0, 0)),
        ],
        out_specs=out_spec,
        out_shape=out_shape,
    )


_mm_l1 = _make_mm_call(2, 64, 512, 4, True)
_mm_l2 = _make_mm_call(4, 128, 512, 4, True)


def _mm3_body(h_ref, w_ref, b_ref, out_ref):
    acc = jnp.broadcast_to(b_ref[...], (256, 256)).astype(jnp.float32)
    for c in range(4):
        acc = acc + jnp.dot(h_ref[c], w_ref[c],
                            preferred_element_type=jnp.float32)
    for q in range(8):
        k, cc = q // 2, q % 2
        out_ref[q] = acc[:, k * 64 + cc * 32: k * 64 + cc * 32 + 32]


_mm_l3 = pl.pallas_call(
    _mm3_body,
    grid=(N_PAD // 256,),
    in_specs=[
        pl.BlockSpec((4, 256, 128), lambda m: (0, m, 0)),
        pl.BlockSpec((4, 128, 256), lambda m: (0, 0, 0)),
        pl.BlockSpec((1, 256), lambda m: (0, 0)),
    ],
    out_specs=pl.BlockSpec((8, 256, 32), lambda m: (0, m, 0)),
    out_shape=_f32((8, N_PAD, 32)),
)


def kernel(x, edge_index, weight, W1, b1, W2, b2, W3, b3):
    f32 = jnp.float32
    i32 = jnp.int32
    pad_e = E_PAD - E
    row2d = jnp.concatenate([edge_index[0], jnp.zeros((pad_e,), i32)]
                            ).reshape(R_PAD, EB)
    col2d = jnp.concatenate([edge_index[1], jnp.zeros((pad_e,), i32)]
                            ).reshape(R_PAD, EB)
    w2d = jnp.concatenate([weight, jnp.zeros((pad_e,), f32)]
                          ).reshape(R_PAD, EB)

    norm2d = _norm_call(row2d, col2d, w2d)

    xp = jnp.pad(x, ((0, N_PAD - N), (0, 0)))
    x3 = xp.reshape(N_PAD, 2, 64).transpose(1, 0, 2)

    h1, h2, h3 = _hop_l1(x3, row2d, col2d, norm2d)
    w1b = W1.reshape(4, 2, 64, 512).reshape(8, 64, 512)
    y1 = _mm_l1(x3, h1, h2, h3, w1b, b1.reshape(1, 512))

    h1, h2, h3 = _hop_l2(y1, row2d, col2d, norm2d)
    w2b = W2.reshape(4, 4, 128, 512).reshape(16, 128, 512)
    y2 = _mm_l2(y1, h1, h2, h3, w2b, b2.reshape(1, 512))

    w3p = jnp.pad(W3, ((0, 0), (0, 0), (0, 64 - 40)))
    w3cat = w3p.transpose(1, 0, 2).reshape(512, 256).reshape(4, 128, 256)
    b3cat = jnp.concatenate([jnp.pad(b3, (0, 64 - 40)),
                             jnp.zeros((192,), f32)]).reshape(1, 256)
    p = _mm_l3(y2, w3cat, b3cat)
    _, _, out = _hop_l3(p, row2d, col2d, norm2d)
    y3 = jnp.concatenate([out[0], out[1]], axis=1)
    return y3[:N, :40]


# speedup vs baseline: 1.1256x; 1.0104x over previous
import functools

import jax
import jax.numpy as jnp
from jax import lax
from jax.experimental import pallas as pl
from jax.experimental.pallas import tpu as pltpu
from jax.experimental.pallas import tpu_sc as plsc

N = 10000
N_PAD = 10240
E = 320000
EB = 128
R = E // EB
R_PAD = 2560
E_PAD = R_PAD * EB
RPT = R_PAD // 16
NPT = N_PAD // 16
SB = 16

_MESH = plsc.VectorSubcoreMesh(core_axis_name="c", subcore_axis_name="s")
_SC_PARAMS = pltpu.CompilerParams(needs_layout_passes=False,
                                  use_tc_tiling_on_sc=False)


def _f32(shape):
    return jax.ShapeDtypeStruct(shape, jnp.float32)


def _norm_body(row_h, col_h, w_h, norm_h, deg_s, dinv_s, colv, wv, rowv,
               normv, dinv_v, zb):
    cid = lax.axis_index("c")
    sid = lax.axis_index("s")
    del cid

    zv = jnp.zeros((16,), jnp.float32)

    def zloop(i, _):
        zb[pl.ds(i * 16, 16)] = zv
        return 0

    lax.fori_loop(0, NPT // 16, zloop, 0)
    pltpu.sync_copy(zb, deg_s.at[pl.ds(sid * NPT, NPT)])
    plsc.subcore_barrier()

    base = sid * RPT
    pltpu.sync_copy(col_h.at[pl.ds(base, RPT)], colv)
    pltpu.sync_copy(w_h.at[pl.ds(base, RPT)], wv)

    def dacc(j, _):
        pltpu.sync_copy(wv.at[j], deg_s.at[colv.at[j]], add=True)
        return 0

    lax.fori_loop(0, RPT, dacc, 0)
    plsc.subcore_barrier()

    nb = sid * NPT
    pltpu.sync_copy(deg_s.at[pl.ds(nb, NPT)], dinv_v.at[pl.ds(0, NPT)])

    def newton(i, _):
        d = dinv_v[pl.ds(i * 16, 16)]
        m = d > 0.0
        ds = jnp.where(m, d, 1.0)
        s = jnp.full((16,), 3.0, jnp.float32)
        for _unused in range(10):
            s = 0.5 * (s + ds / s)
        dinv_v[pl.ds(i * 16, 16)] = jnp.where(m, 1.0 / s, 0.0)
        return 0

    lax.fori_loop(0, NPT // 16, newton, 0)
    pltpu.sync_copy(dinv_v.at[pl.ds(0, NPT)], dinv_s.at[pl.ds(nb, NPT)])
    plsc.subcore_barrier()

    wid = sid * 2 + lax.axis_index("c")
    rows_w = R_PAD // 32
    eb = wid * rows_w
    pltpu.sync_copy(dinv_s, dinv_v)
    pltpu.sync_copy(row_h.at[pl.ds(eb, rows_w)], rowv)
    pltpu.sync_copy(col_h.at[pl.ds(eb, rows_w)], colv.at[pl.ds(0, rows_w)])
    pltpu.sync_copy(w_h.at[pl.ds(eb, rows_w)], wv.at[pl.ds(0, rows_w)])

    def crow(j, _):
        def c16(v, _2):
            r_idx = rowv[j, pl.ds(v * 16, 16)]
            c_idx = colv[j, pl.ds(v * 16, 16)]
            w16 = wv[j, pl.ds(v * 16, 16)]
            rd = plsc.load_gather(dinv_v, [r_idx])
            cd = plsc.load_gather(dinv_v, [c_idx])
            normv[j, pl.ds(v * 16, 16)] = rd * w16 * cd
            return 0

        lax.fori_loop(0, EB // 16, c16, 0)
        return 0

    lax.fori_loop(0, rows_w, crow, 0)
    pltpu.sync_copy(normv, norm_h.at[pl.ds(eb, rows_w)])


_norm_call = pl.kernel(
    _norm_body,
    out_type=_f32((R_PAD, EB)),
    mesh=_MESH,
    scratch_types=[
        pltpu.VMEM_SHARED((N_PAD,), jnp.float32),
        pltpu.VMEM_SHARED((N_PAD,), jnp.float32),
        pltpu.VMEM((RPT, EB), jnp.int32),
        pltpu.VMEM((RPT, EB), jnp.float32),
        pltpu.VMEM((R_PAD // 32, EB), jnp.int32),
        pltpu.VMEM((R_PAD // 32, EB), jnp.float32),
        pltpu.VMEM((N_PAD,), jnp.float32),
        pltpu.VMEM((NPT,), jnp.float32),
    ],
    compiler_params=_SC_PARAMS,
)


def _hop_body(nc, fc, horner, nb, sb_rows, h_in, row_h, col_h, norm_h, o1, o2, o3,
              acc, rowv, colv, normv, *bufsems):
    cid = lax.axis_index("c")
    sid = lax.axis_index("s")
    nch = nc // 2
    nvr = fc // 16
    outs = [o1, o2, o3]
    gbufs = bufsems[:nb]
    gsems = bufsems[nb:2 * nb]
    ssems = bufsems[2 * nb:3 * nb]

    def start_gather(src, j, b):
        pltpu.async_copy(src.at[rowv.at[j]], gbufs[b], gsems[b])

    def wait_gather(src, b):
        pltpu.make_async_copy(src.at[pl.ds(0, EB)], gbufs[b],
                              gsems[b]).wait()

    def start_scatter(j, b):
        pltpu.async_copy(gbufs[b], acc.at[colv.at[j]], ssems[b], add=True)

    def wait_scatter(b):
        pltpu.make_async_copy(gbufs[b], acc.at[pl.ds(0, EB)],
                              ssems[b]).wait()

    def scale(j, b):
        gbuf = gbufs[b]

        def scale16(e16, _3):
            nv = normv[j, pl.ds(e16 * 16, 16)]
            for e in range(16):
                edge = e16 * 16 + e
                s = nv[e]
                for v in range(nvr):
                    g = gbuf[edge, pl.ds(v * 16, 16)]
                    gbuf[edge, pl.ds(v * 16, 16)] = g * s
            return 0

        lax.fori_loop(0, EB // 16, scale16, 0)

    def do_chunk(ci, _):
        c = cid * nch + ci
        for k in range(3):
            if horner:
                src = h_in.at[3 * nc + c] if k == 0 else outs[k - 1].at[c]
            else:
                src = h_in.at[c] if k == 0 else outs[k - 1].at[c]
            dst = outs[k]

            if horner:
                pltpu.sync_copy(
                    h_in.at[(2 - k) * nc + c].at[pl.ds(sid * NPT, NPT)],
                    acc.at[pl.ds(sid * NPT, NPT)])
            else:
                zv = jnp.zeros((16,), jnp.float32)
                zbuf = gbufs[0]

                def zrow(r, _2):
                    for v in range(nvr):
                        zbuf[r, pl.ds(v * 16, 16)] = zv
                    return 0

                lax.fori_loop(0, EB, zrow, 0)
                for i in range(NPT // EB):
                    pltpu.sync_copy(zbuf,
                                    acc.at[pl.ds(sid * NPT + i * EB, EB)])
            plsc.subcore_barrier()

            base = sid * RPT

            def sub_block(sb, _1):
                off = base + sb * sb_rows
                pltpu.sync_copy(row_h.at[pl.ds(off, sb_rows)], rowv)
                pltpu.sync_copy(col_h.at[pl.ds(off, sb_rows)], colv)
                pltpu.sync_copy(norm_h.at[pl.ds(off, sb_rows)], normv)
                for b in range(nb):
                    start_gather(src, b, b)

                def group(p, _2):
                    for b in range(nb):
                        j = p * nb + b
                        wait_gather(src, b)
                        scale(j, b)
                        start_scatter(j, b)

                    @pl.when(p < sb_rows // nb - 1)
                    def _prefetch():
                        for b in range(nb):
                            wait_scatter(b)
                            start_gather(src, p * nb + nb + b, b)

                    return 0

                lax.fori_loop(0, sb_rows // nb, group, 0)
                for b in range(nb):
                    wait_scatter(b)
                return 0

            lax.fori_loop(0, RPT // sb_rows, sub_block, 0)
            plsc.subcore_barrier()
            pltpu.sync_copy(acc.at[pl.ds(sid * NPT, NPT)],
                            dst.at[c].at[pl.ds(sid * NPT, NPT)])
            plsc.subcore_barrier()
        return 0

    lax.fori_loop(0, nch, do_chunk, 0)


def _make_hop_call(nc, fc, horner=False, nb=2, sb_rows=SB):
    body = functools.partial(_hop_body, nc, fc, horner, nb, sb_rows)
    return pl.kernel(
        body,
        out_type=[_f32((nc, N_PAD, fc))] * 3,
        mesh=_MESH,
        scratch_types=[
            pltpu.VMEM_SHARED((N_PAD, fc), jnp.float32),
            pltpu.VMEM((sb_rows, EB), jnp.int32),
            pltpu.VMEM((sb_rows, EB), jnp.int32),
            pltpu.VMEM((sb_rows, EB), jnp.float32),
        ] + [pltpu.VMEM((EB, fc), jnp.float32)] * nb
          + [pltpu.SemaphoreType.DMA] * (2 * nb),
        compiler_params=_SC_PARAMS,
    )


_hop_l1 = _make_hop_call(2, 64, nb=4, sb_rows=32)
_hop_l2 = _make_hop_call(4, 128, nb=2)
_hop_l3 = _make_hop_call(2, 32, horner=True, nb=4, sb_rows=32)


def _mm_body(n_in, nc_out, elu, *refs):
    h_refs = refs[:4]
    w_ref, b_ref, out_ref = refs[4], refs[5], refs[6]
    nch_in = n_in // 4
    acc = jnp.broadcast_to(b_ref[...], (256, b_ref.shape[1])).astype(jnp.float32)
    for k in range(4):
        for c in range(nch_in):
            acc = acc + jnp.dot(h_refs[k][c], w_ref[k * nch_in + c],
                                preferred_element_type=jnp.float32)
    if elu:
        acc = jnp.where(acc > 0, acc, jnp.exp(acc) - 1.0)
    if nc_out > 1:
        fc = acc.shape[1] // nc_out
        for c in range(nc_out):
            out_ref[c] = acc[:, c * fc:(c + 1) * fc]
    else:
        out_ref[...] = acc


def _make_mm_call(nc_in, fc_in, h_out, nc_out, elu):
    n_in = 4 * nc_in
    body = functools.partial(_mm_body, n_in, nc_out, elu)
    if nc_out > 1:
        out_shape = _f32((nc_out, N_PAD, h_out // nc_out))
        out_spec = pl.BlockSpec((nc_out, 256, h_out // nc_out),
                                lambda m: (0, m, 0))
    else:
        out_shape = _f32((N_PAD, h_out))
        out_spec = pl.BlockSpec((256, h_out), lambda m: (m, 0))
    h_spec = pl.BlockSpec((nc_in, 256, fc_in), lambda m: (0, m, 0))
    return pl.pallas_call(
        body,
        grid=(N_PAD // 256,),
        in_specs=[h_spec] * 4 + [
            pl.BlockSpec((n_in, fc_in, h_out), lambda m: (0, 0, 0)),
            pl.BlockSpec((1, h_out), lambda m: (0, 0)),
        ],
        out_specs=out_spec,
        out_shape=out_shape,
    )


_mm_l1 = _make_mm_call(2, 64, 512, 4, True)
_mm_l2 = _make_mm_call(4, 128, 512, 4, True)


def _mm3_body(h_ref, w_ref, b_ref, out_ref):
    acc = jnp.broadcast_to(b_ref[...], (256, 256)).astype(jnp.float32)
    for c in range(4):
        acc = acc + jnp.dot(h_ref[c], w_ref[c],
                            preferred_element_type=jnp.float32)
    for q in range(8):
        k, cc = q // 2, q % 2
        out_ref[q] = acc[:, k * 64 + cc * 32: k * 64 + cc * 32 + 32]


_mm_l3 = pl.pallas_call(
    _mm3_body,
    grid=(N_PAD // 256,),
    in_specs=[
        pl.BlockSpec((4, 256, 128), lambda m: (0, m, 0)),
        pl.BlockSpec((4, 128, 256), lambda m: (0, 0, 0)),
        pl.BlockSpec((1, 256), lambda m: (0, 0)),
    ],
    out_specs=pl.BlockSpec((8, 256, 32), lambda m: (0, m, 0)),
    out_shape=_f32((8, N_PAD, 32)),
)


def kernel(x, edge_index, weight, W1, b1, W2, b2, W3, b3):
    f32 = jnp.float32
    i32 = jnp.int32
    pad_e = E_PAD - E
    row2d = jnp.concatenate([edge_index[0], jnp.zeros((pad_e,), i32)]
                            ).reshape(R_PAD, EB)
    col2d = jnp.concatenate([edge_index[1], jnp.zeros((pad_e,), i32)]
                            ).reshape(R_PAD, EB)
    w2d = jnp.concatenate([weight, jnp.zeros((pad_e,), f32)]
                          ).reshape(R_PAD, EB)

    norm2d = _norm_call(row2d, col2d, w2d)

    xp = jnp.pad(x, ((0, N_PAD - N), (0, 0)))
    x3 = xp.reshape(N_PAD, 2, 64).transpose(1, 0, 2)

    h1, h2, h3 = _hop_l1(x3, row2d, col2d, norm2d)
    w1b = W1.reshape(4, 2, 64, 512).reshape(8, 64, 512)
    y1 = _mm_l1(x3, h1, h2, h3, w1b, b1.reshape(1, 512))

    h1, h2, h3 = _hop_l2(y1, row2d, col2d, norm2d)
    w2b = W2.reshape(4, 4, 128, 512).reshape(16, 128, 512)
    y2 = _mm_l2(y1, h1, h2, h3, w2b, b2.reshape(1, 512))

    w3p = jnp.pad(W3, ((0, 0), (0, 0), (0, 64 - 40)))
    w3cat = w3p.transpose(1, 0, 2).reshape(512, 256).reshape(4, 128, 256)
    b3cat = jnp.concatenate([jnp.pad(b3, (0, 64 - 40)),
                             jnp.zeros((192,), f32)]).reshape(1, 256)
    p = _mm_l3(y2, w3cat, b3cat)
    _, _, out = _hop_l3(p, row2d, col2d, norm2d)
    y3 = jnp.concatenate([out[0], out[1]], axis=1)
    return y3[:N, :40]
